# baseline (speedup 1.0000x reference)
import functools

import jax
import jax.numpy as jnp
from jax import lax
from jax.experimental import pallas as pl
from jax.experimental.pallas import tpu as pltpu
from jax.experimental.pallas import tpu_sc as plsc

NUM_EXPERT = 8
IN_FEAT = 1024
OUT_FEAT = 4096
N_TOK = 4096

A = 128
A_SHIFT = 7
T_TILES = N_TOK // A + NUM_EXPERT
M_PAD = T_TILES * A
BN = 4096
N_BLOCKS = OUT_FEAT // BN

NW = 32
CHUNK = N_TOK // NW
NVEC = N_TOK // 16


def _sc_mesh():
    return plsc.VectorSubcoreMesh(core_axis_name="c", subcore_axis_name="s")


def _route_body(inp_hbm, gate_hbm, xs_hbm, pos_hbm, gids_hbm, ntiles_hbm,
                gate_v, pos_buf, idx2d, rowbuf, rowbuf2, gids_buf, nt_buf,
                ls0, ls1, ss0, ss1):
    wid = lax.axis_index("s") * 2 + lax.axis_index("c")
    lanes = lax.iota(jnp.int32, 16)
    zero16 = jnp.zeros((16,), jnp.int32)

    pltpu.sync_copy(gate_hbm, gate_v)

    my_first_vec = wid * (CHUNK // 16)

    def p1_body(i, carry):
        hist, prefix = carry
        g = gate_v[pl.ds(i * 16, 16)]
        prefix = jnp.where(i == my_first_vec, hist, prefix)
        for e in range(NUM_EXPERT):
            cnt = jnp.sum(jnp.where(g == e, jnp.int32(1), jnp.int32(0)))
            hist = hist + jnp.where(lanes == e, jnp.full((16,), cnt, jnp.int32),
                                    zero16)
        return hist, prefix

    hist, prefix = lax.fori_loop(0, NVEC, p1_body, (zero16, zero16))

    cum_tiles = jnp.int32(0)
    base = []
    cum_list = []
    for e in range(NUM_EXPERT):
        tot_e = hist[e]
        tiles_e = lax.shift_right_logical(tot_e + (A - 1), A_SHIFT)
        start_e = cum_tiles * A
        base.append(jnp.full((16,), start_e + prefix[e], jnp.int32))
        cum_tiles = cum_tiles + tiles_e
        cum_list.append(cum_tiles)

    @pl.when(wid == 0)
    def _():
        for half in range(3):
            t_vec = lanes + half * 16
            g_t = zero16
            for e in range(NUM_EXPERT - 1):
                g_t = g_t + jnp.where(t_vec >= cum_list[e],
                                      jnp.int32(1), jnp.int32(0))
            gids_buf[pl.ds(half * 16, 16)] = g_t
        pltpu.sync_copy(gids_buf, gids_hbm)
        nt_buf[...] = jnp.full((16,), cum_tiles, jnp.int32)
        pltpu.sync_copy(nt_buf, ntiles_hbm)

    for v in range(CHUNK // 16):
        g = gate_v[pl.ds((wid * (CHUNK // 16) + v) * 16, 16)]
        posv = zero16
        for e in range(NUM_EXPERT):
            m = g == e
            ones = jnp.where(m, jnp.int32(1), jnp.int32(0))
            cs = plsc.cumsum(ones)
            posv = posv + jnp.where(m, base[e] + cs - 1, zero16)
            base[e] = base[e] + jnp.full((16,), cs[15], jnp.int32)
        pos_buf[pl.ds(v * 16, 16)] = posv
        idx2d[v // 2, pl.ds((v % 2) * 16, 16)] = posv

    pltpu.sync_copy(pos_buf, pos_hbm.at[pl.ds(wid * CHUNK, CHUNK)])

    rowbufs = (rowbuf, rowbuf2)
    lsems, ssems = (ls0, ls1), (ss0, ss1)
    lins = [None, None]
    scats = [None, None]

    def scat(b):
        j = b % 2
        lins[j].wait()
        scats[j] = pltpu.async_copy(rowbufs[j], xs_hbm.at[idx2d.at[b]], ssems[j])

    for b in range(4):
        i = b % 2
        if scats[i] is not None:
            scats[i].wait()
        r0 = wid * CHUNK + b * 32
        lins[i] = pltpu.async_copy(inp_hbm.at[pl.ds(r0, 32)], rowbufs[i], lsems[i])
        if b > 0:
            scat(b - 1)
    scat(3)
    scats[0].wait()
    scats[1].wait()


def _route_sc(inp, gate):
    kfn = pl.kernel(
        _route_body, mesh=_sc_mesh(),
        compiler_params=pltpu.CompilerParams(needs_layout_passes=False),
        out_type=[
            jax.ShapeDtypeStruct((M_PAD, IN_FEAT), jnp.float32),
            jax.ShapeDtypeStruct((N_TOK,), jnp.int32),
            jax.ShapeDtypeStruct((48,), jnp.int32),
            jax.ShapeDtypeStruct((16,), jnp.int32),
        ],
        scratch_types=[
            pltpu.VMEM((N_TOK,), jnp.int32),
            pltpu.VMEM((CHUNK,), jnp.int32),
            pltpu.VMEM((4, 32), jnp.int32),
            pltpu.VMEM((32, IN_FEAT), jnp.float32),
            pltpu.VMEM((32, IN_FEAT), jnp.float32),
            pltpu.VMEM((48,), jnp.int32),
            pltpu.VMEM((16,), jnp.int32),
            pltpu.SemaphoreType.DMA,
            pltpu.SemaphoreType.DMA,
            pltpu.SemaphoreType.DMA,
            pltpu.SemaphoreType.DMA,
        ],
    )
    return kfn(inp, gate)


def _mm_body(gids_ref, nt_ref, xs_ref, w_ref, ys_ref):
    t = pl.program_id(1)

    @pl.when(t < nt_ref[0])
    def _():
        ys_ref[...] = lax.dot_general(
            xs_ref[...], w_ref[0],
            dimension_numbers=(((1,), (1,)), ((), ())),
            preferred_element_type=jnp.float32)


def _grouped_matmul(xs, weight, gids, ntiles):
    return pl.pallas_call(
        _mm_body,
        grid_spec=pltpu.PrefetchScalarGridSpec(
            num_scalar_prefetch=2,
            grid=(N_BLOCKS, T_TILES),
            in_specs=[
                pl.BlockSpec((A, IN_FEAT),
                             lambda n, t, gids, nt: (jnp.minimum(t, nt[0] - 1), 0)),
                pl.BlockSpec((1, BN, IN_FEAT),
                             lambda n, t, gids, nt:
                             (gids[jnp.minimum(t, nt[0] - 1)], n, 0)),
            ],
            out_specs=pl.BlockSpec(
                (A, BN),
                lambda n, t, gids, nt: (jnp.minimum(t, nt[0] - 1), n)),
        ),
        out_shape=jax.ShapeDtypeStruct((M_PAD, OUT_FEAT), jnp.float32),
    )(gids, ntiles, xs, weight)


RB = 8


def _unperm_body(ys_hbm, pos_hbm, out_hbm, posbuf,
                 buf0, buf1, buf2, gs0, gs1, gs2, ws0, ws1, ws2):
    wid = lax.axis_index("s") * 2 + lax.axis_index("c")
    pltpu.sync_copy(pos_hbm.at[pl.ds(wid * CHUNK, CHUNK)], posbuf)
    bufs, gsems, wsems = (buf0, buf1, buf2), (gs0, gs1, gs2), (ws0, ws1, ws2)
    nb = CHUNK // RB
    gathers = [None, None, None]
    wbacks = [None, None, None]

    def wb(b):
        j = b % 3
        gathers[j].wait()
        wbacks[j] = pltpu.async_copy(
            bufs[j], out_hbm.at[pl.ds(wid * CHUNK + b * RB, RB)], wsems[j])

    for b in range(nb):
        i = b % 3
        if wbacks[i] is not None:
            wbacks[i].wait()
        gathers[i] = pltpu.async_copy(
            ys_hbm.at[posbuf.at[pl.ds(b * RB, RB)]], bufs[i], gsems[i])
        if b > 0:
            wb(b - 1)
    wb(nb - 1)
    for j in range(3):
        if wbacks[j] is not None:
            wbacks[j].wait()


def _unpermute_sc(ys, pos):
    kfn = pl.kernel(
        _unperm_body, mesh=_sc_mesh(),
        compiler_params=pltpu.CompilerParams(needs_layout_passes=False),
        out_type=jax.ShapeDtypeStruct((N_TOK, OUT_FEAT), jnp.float32),
        scratch_types=[
            pltpu.VMEM((CHUNK,), jnp.int32),
            pltpu.VMEM((RB, OUT_FEAT), jnp.float32),
            pltpu.VMEM((RB, OUT_FEAT), jnp.float32),
            pltpu.VMEM((RB, OUT_FEAT), jnp.float32),
            pltpu.SemaphoreType.DMA,
            pltpu.SemaphoreType.DMA,
            pltpu.SemaphoreType.DMA,
            pltpu.SemaphoreType.DMA,
            pltpu.SemaphoreType.DMA,
            pltpu.SemaphoreType.DMA,
        ],
    )
    return kfn(ys, pos)


def kernel(inp, gate, weight):
    xs, pos, gids, ntiles = _route_sc(inp, gate)
    ys = _grouped_matmul(xs, weight, gids, ntiles)
    return _unpermute_sc(ys, pos)


# speedup vs baseline: 1.2448x; 1.2448x over previous
import functools

import jax
import jax.numpy as jnp
from jax import lax
from jax.experimental import pallas as pl
from jax.experimental.pallas import tpu as pltpu
from jax.experimental.pallas import tpu_sc as plsc

NUM_EXPERT = 8
IN_FEAT = 1024
OUT_FEAT = 4096
N_TOK = 4096

A = 256
T_TILES = N_TOK // A + NUM_EXPERT
M_PAD = T_TILES * A
BN = 4096
N_BLOCKS = OUT_FEAT // BN

NW = 32
CHUNK = N_TOK // NW
NVEC = N_TOK // 16


def _sc_mesh():
    return plsc.VectorSubcoreMesh(core_axis_name="c", subcore_axis_name="s")


def _route_body(inp_hbm, gate_hbm, xs_hbm, pos_hbm, gids_hbm, ntiles_hbm,
                gate_v, pos_buf, idx2d, rowbuf, rowbuf2, gids_buf, nt_buf,
                ls0, ls1, ss0, ss1):
    wid = lax.axis_index("s") * 2 + lax.axis_index("c")
    lanes = lax.iota(jnp.int32, 16)
    zero16 = jnp.zeros((16,), jnp.int32)

    rowbufs = (rowbuf, rowbuf2)
    lsems, ssems = (ls0, ls1), (ss0, ss1)
    lins = [None, None]
    scats = [None, None]
    for b in range(2):
        r0 = wid * CHUNK + b * 32
        lins[b] = pltpu.async_copy(inp_hbm.at[pl.ds(r0, 32)], rowbufs[b],
                                   lsems[b])

    pltpu.sync_copy(gate_hbm, gate_v)

    my_first_vec = wid * (CHUNK // 16)

    def p1_body(i, carry):
        hist, prefix = carry
        g = gate_v[pl.ds(i * 16, 16)]
        prefix = jnp.where(i == my_first_vec, hist, prefix)
        for e in range(NUM_EXPERT):
            cnt = jnp.sum(jnp.where(g == e, jnp.int32(1), jnp.int32(0)))
            hist = hist + jnp.where(lanes == e, jnp.full((16,), cnt, jnp.int32),
                                    zero16)
        return hist, prefix

    hist, prefix = lax.fori_loop(0, NVEC, p1_body, (zero16, zero16))

    cum_tiles = jnp.int32(0)
    base = []
    cum_list = []
    for e in range(NUM_EXPERT):
        tot_e = hist[e]
        tiles_e = lax.shift_right_logical(tot_e + (A - 1), 8)
        start_e = cum_tiles * A
        base.append(jnp.full((16,), start_e + prefix[e], jnp.int32))
        cum_tiles = cum_tiles + tiles_e
        cum_list.append(cum_tiles)

    @pl.when(wid == 0)
    def _():
        for half in range(2):
            t_vec = lanes + half * 16
            g_t = zero16
            for e in range(NUM_EXPERT - 1):
                g_t = g_t + jnp.where(t_vec >= cum_list[e],
                                      jnp.int32(1), jnp.int32(0))
            gids_buf[pl.ds(half * 16, 16)] = g_t
        pltpu.sync_copy(gids_buf, gids_hbm)
        nt_buf[...] = jnp.full((16,), cum_tiles, jnp.int32)
        pltpu.sync_copy(nt_buf, ntiles_hbm)

    for v in range(CHUNK // 16):
        g = gate_v[pl.ds((wid * (CHUNK // 16) + v) * 16, 16)]
        posv = zero16
        for e in range(NUM_EXPERT):
            m = g == e
            ones = jnp.where(m, jnp.int32(1), jnp.int32(0))
            cs = plsc.cumsum(ones)
            posv = posv + jnp.where(m, base[e] + cs - 1, zero16)
            base[e] = base[e] + jnp.full((16,), cs[15], jnp.int32)
        pos_buf[pl.ds(v * 16, 16)] = posv
        idx2d[v // 2, pl.ds((v % 2) * 16, 16)] = posv

    pltpu.sync_copy(pos_buf, pos_hbm.at[pl.ds(wid * CHUNK, CHUNK)])

    def scat(b):
        j = b % 2
        lins[j].wait()
        scats[j] = pltpu.async_copy(rowbufs[j], xs_hbm.at[idx2d.at[b]], ssems[j])

    scat(0)
    scat(1)
    for b in range(2, 4):
        i = b % 2
        scats[i].wait()
        r0 = wid * CHUNK + b * 32
        lins[i] = pltpu.async_copy(inp_hbm.at[pl.ds(r0, 32)], rowbufs[i],
                                   lsems[i])
    scat(2)
    scat(3)
    scats[0].wait()
    scats[1].wait()


def _route_sc(inp, gate):
    kfn = pl.kernel(
        _route_body, mesh=_sc_mesh(),
        compiler_params=pltpu.CompilerParams(needs_layout_passes=False),
        out_type=[
            jax.ShapeDtypeStruct((M_PAD, IN_FEAT), jnp.float32),
            jax.ShapeDtypeStruct((N_TOK,), jnp.int32),
            jax.ShapeDtypeStruct((32,), jnp.int32),
            jax.ShapeDtypeStruct((16,), jnp.int32),
        ],
        scratch_types=[
            pltpu.VMEM((N_TOK,), jnp.int32),
            pltpu.VMEM((CHUNK,), jnp.int32),
            pltpu.VMEM((4, 32), jnp.int32),
            pltpu.VMEM((32, IN_FEAT), jnp.float32),
            pltpu.VMEM((32, IN_FEAT), jnp.float32),
            pltpu.VMEM((32,), jnp.int32),
            pltpu.VMEM((16,), jnp.int32),
            pltpu.SemaphoreType.DMA,
            pltpu.SemaphoreType.DMA,
            pltpu.SemaphoreType.DMA,
            pltpu.SemaphoreType.DMA,
        ],
    )
    return kfn(inp, gate)


def _mm_body(gids_ref, nt_ref, xs_ref, w_ref, ys_ref):
    t = pl.program_id(1)

    @pl.when(t < nt_ref[0])
    def _():
        ys_ref[...] = lax.dot_general(
            xs_ref[...], w_ref[0],
            dimension_numbers=(((1,), (1,)), ((), ())),
            preferred_element_type=jnp.float32)


def _grouped_matmul(xs, weight, gids, ntiles):
    return pl.pallas_call(
        _mm_body,
        grid_spec=pltpu.PrefetchScalarGridSpec(
            num_scalar_prefetch=2,
            grid=(N_BLOCKS, T_TILES),
            in_specs=[
                pl.BlockSpec((A, IN_FEAT),
                             lambda n, t, gids, nt: (jnp.minimum(t, nt[0] - 1), 0)),
                pl.BlockSpec((1, BN, IN_FEAT),
                             lambda n, t, gids, nt:
                             (gids[jnp.minimum(t, nt[0] - 1)], n, 0)),
            ],
            out_specs=pl.BlockSpec(
                (A, BN),
                lambda n, t, gids, nt: (jnp.minimum(t, nt[0] - 1), n)),
        ),
        out_shape=jax.ShapeDtypeStruct((M_PAD, OUT_FEAT), jnp.float32),
    )(gids, ntiles, xs, weight)


RB = 8


def _unperm_body(ys_hbm, pos_hbm, out_hbm, posbuf,
                 buf0, buf1, buf2, gs0, gs1, gs2, ws0, ws1, ws2):
    wid = lax.axis_index("s") * 2 + lax.axis_index("c")
    pltpu.sync_copy(pos_hbm.at[pl.ds(wid * CHUNK, CHUNK)], posbuf)
    bufs, gsems, wsems = (buf0, buf1, buf2), (gs0, gs1, gs2), (ws0, ws1, ws2)
    nb = CHUNK // RB
    gathers = [None, None, None]
    wbacks = [None, None, None]

    def wb(b):
        j = b % 3
        gathers[j].wait()
        wbacks[j] = pltpu.async_copy(
            bufs[j], out_hbm.at[pl.ds(wid * CHUNK + b * RB, RB)], wsems[j])

    for b in range(nb):
        i = b % 3
        if wbacks[i] is not None:
            wbacks[i].wait()
        gathers[i] = pltpu.async_copy(
            ys_hbm.at[posbuf.at[pl.ds(b * RB, RB)]], bufs[i], gsems[i])
        if b > 0:
            wb(b - 1)
    wb(nb - 1)
    for j in range(3):
        if wbacks[j] is not None:
            wbacks[j].wait()


def _unpermute_sc(ys, pos):
    kfn = pl.kernel(
        _unperm_body, mesh=_sc_mesh(),
        compiler_params=pltpu.CompilerParams(needs_layout_passes=False),
        out_type=jax.ShapeDtypeStruct((N_TOK, OUT_FEAT), jnp.float32),
        scratch_types=[
            pltpu.VMEM((CHUNK,), jnp.int32),
            pltpu.VMEM((RB, OUT_FEAT), jnp.float32),
            pltpu.VMEM((RB, OUT_FEAT), jnp.float32),
            pltpu.VMEM((RB, OUT_FEAT), jnp.float32),
            pltpu.SemaphoreType.DMA,
            pltpu.SemaphoreType.DMA,
            pltpu.SemaphoreType.DMA,
            pltpu.SemaphoreType.DMA,
            pltpu.SemaphoreType.DMA,
            pltpu.SemaphoreType.DMA,
        ],
    )
    return kfn(ys, pos)


def kernel(inp, gate, weight):
    xs, pos, gids, ntiles = _route_sc(inp, gate)
    ys = _grouped_matmul(xs, weight, gids, ntiles)
    return _unpermute_sc(ys, pos)


# speedup vs baseline: 1.2460x; 1.0010x over previous
import jax
import jax.numpy as jnp
from jax import lax
from jax.experimental import pallas as pl
from jax.experimental.pallas import tpu as pltpu
from jax.experimental.pallas import tpu_sc as plsc

NUM_EXPERT = 8
IN_FEAT = 1024
OUT_FEAT = 4096
N_TOK = 4096

A = 256
T_TILES = N_TOK // A + NUM_EXPERT
M_PAD = T_TILES * A
BN = 4096
N_BLOCKS = OUT_FEAT // BN

NW = 32
CHUNK = N_TOK // NW
NVEC = N_TOK // 16


def _sc_mesh():
    return plsc.VectorSubcoreMesh(core_axis_name="c", subcore_axis_name="s")


def _route_body(inp_hbm, gate_hbm, xs_hbm, pos_hbm, gids_hbm, ntiles_hbm,
                gate_v, pos_buf, idx2d, rowbuf, rowbuf2, gids_buf, nt_buf,
                ls0, ls1, ss0, ss1):
    wid = lax.axis_index("s") * 2 + lax.axis_index("c")
    lanes = lax.iota(jnp.int32, 16)
    zero16 = jnp.zeros((16,), jnp.int32)

    rowbufs = (rowbuf, rowbuf2)
    lsems, ssems = (ls0, ls1), (ss0, ss1)
    lins = [None, None]
    scats = [None, None]
    for b in range(2):
        r0 = wid * CHUNK + b * 32
        lins[b] = pltpu.async_copy(inp_hbm.at[pl.ds(r0, 32)], rowbufs[b],
                                   lsems[b])

    pltpu.sync_copy(gate_hbm, gate_v)

    my_first_vec = wid * (CHUNK // 16)

    def p1_body(i, carry):
        hist, prefix = carry
        g = gate_v[pl.ds(i * 16, 16)]
        prefix = jnp.where(i == my_first_vec, hist, prefix)
        for e in range(NUM_EXPERT):
            cnt = jnp.sum(jnp.where(g == e, jnp.int32(1), jnp.int32(0)))
            hist = hist + jnp.where(lanes == e, jnp.full((16,), cnt, jnp.int32),
                                    zero16)
        return hist, prefix

    hist, prefix = lax.fori_loop(0, NVEC, p1_body, (zero16, zero16))

    cum_tiles = jnp.int32(0)
    base = []
    cum_list = []
    for e in range(NUM_EXPERT):
        tot_e = hist[e]
        tiles_e = lax.shift_right_logical(tot_e + (A - 1), 8)
        start_e = cum_tiles * A
        base.append(jnp.full((16,), start_e + prefix[e], jnp.int32))
        cum_tiles = cum_tiles + tiles_e
        cum_list.append(cum_tiles)

    @pl.when(wid == 0)
    def _():
        for half in range(2):
            t_vec = lanes + half * 16
            g_t = zero16
            for e in range(NUM_EXPERT - 1):
                g_t = g_t + jnp.where(t_vec >= cum_list[e],
                                      jnp.int32(1), jnp.int32(0))
            gids_buf[pl.ds(half * 16, 16)] = g_t
        pltpu.sync_copy(gids_buf, gids_hbm)
        nt_buf[...] = jnp.full((16,), cum_tiles, jnp.int32)
        pltpu.sync_copy(nt_buf, ntiles_hbm)

    for v in range(CHUNK // 16):
        g = gate_v[pl.ds((wid * (CHUNK // 16) + v) * 16, 16)]
        posv = zero16
        for e in range(NUM_EXPERT):
            m = g == e
            ones = jnp.where(m, jnp.int32(1), jnp.int32(0))
            cs = plsc.cumsum(ones)
            posv = posv + jnp.where(m, base[e] + cs - 1, zero16)
            base[e] = base[e] + jnp.full((16,), cs[15], jnp.int32)
        pos_buf[pl.ds(v * 16, 16)] = posv
        idx2d[v // 2, pl.ds((v % 2) * 16, 16)] = posv

    pltpu.sync_copy(pos_buf, pos_hbm.at[pl.ds(wid * CHUNK, CHUNK)])

    def scat(b):
        j = b % 2
        lins[j].wait()
        scats[j] = pltpu.async_copy(rowbufs[j], xs_hbm.at[idx2d.at[b]], ssems[j])

    scat(0)
    scat(1)
    for b in range(2, 4):
        i = b % 2
        scats[i].wait()
        r0 = wid * CHUNK + b * 32
        lins[i] = pltpu.async_copy(inp_hbm.at[pl.ds(r0, 32)], rowbufs[i],
                                   lsems[i])
    scat(2)
    scat(3)
    scats[0].wait()
    scats[1].wait()


def _route_sc(inp, gate):
    kfn = pl.kernel(
        _route_body, mesh=_sc_mesh(),
        compiler_params=pltpu.CompilerParams(needs_layout_passes=False),
        out_type=[
            jax.ShapeDtypeStruct((M_PAD, IN_FEAT), jnp.float32),
            jax.ShapeDtypeStruct((N_TOK,), jnp.int32),
            jax.ShapeDtypeStruct((32,), jnp.int32),
            jax.ShapeDtypeStruct((16,), jnp.int32),
        ],
        scratch_types=[
            pltpu.VMEM((N_TOK,), jnp.int32),
            pltpu.VMEM((CHUNK,), jnp.int32),
            pltpu.VMEM((4, 32), jnp.int32),
            pltpu.VMEM((32, IN_FEAT), jnp.float32),
            pltpu.VMEM((32, IN_FEAT), jnp.float32),
            pltpu.VMEM((32,), jnp.int32),
            pltpu.VMEM((16,), jnp.int32),
            pltpu.SemaphoreType.DMA,
            pltpu.SemaphoreType.DMA,
            pltpu.SemaphoreType.DMA,
            pltpu.SemaphoreType.DMA,
        ],
    )
    return kfn(inp, gate)


def _mm_body(gids_ref, nt_ref, xs_ref, w_ref, ys_ref):
    t = pl.program_id(1)

    @pl.when(t < nt_ref[0])
    def _():
        ys_ref[...] = lax.dot_general(
            xs_ref[...], w_ref[0],
            dimension_numbers=(((1,), (1,)), ((), ())),
            preferred_element_type=jnp.float32)


def _grouped_matmul(xs, weight, gids, ntiles):
    return pl.pallas_call(
        _mm_body,
        grid_spec=pltpu.PrefetchScalarGridSpec(
            num_scalar_prefetch=2,
            grid=(N_BLOCKS, T_TILES),
            in_specs=[
                pl.BlockSpec((A, IN_FEAT),
                             lambda n, t, gids, nt: (jnp.minimum(t, nt[0] - 1), 0)),
                pl.BlockSpec((1, BN, IN_FEAT),
                             lambda n, t, gids, nt:
                             (gids[jnp.minimum(t, nt[0] - 1)], n, 0)),
            ],
            out_specs=pl.BlockSpec(
                (A, BN),
                lambda n, t, gids, nt: (jnp.minimum(t, nt[0] - 1), n)),
        ),
        out_shape=jax.ShapeDtypeStruct((M_PAD, OUT_FEAT), jnp.float32),
    )(gids, ntiles, xs, weight)


RB = 8


def _unperm_body(ys_hbm, pos_hbm, out_hbm, posbuf,
                 buf0, buf1, buf2, gs0, gs1, gs2, ws0, ws1, ws2):
    wid = lax.axis_index("s") * 2 + lax.axis_index("c")
    pltpu.sync_copy(pos_hbm.at[pl.ds(wid * CHUNK, CHUNK)], posbuf)
    bufs, gsems, wsems = (buf0, buf1, buf2), (gs0, gs1, gs2), (ws0, ws1, ws2)
    nb = CHUNK // RB
    gathers = [None, None, None]
    wbacks = [None, None, None]

    def wb(b):
        j = b % 3
        gathers[j].wait()
        wbacks[j] = pltpu.async_copy(
            bufs[j], out_hbm.at[pl.ds(wid * CHUNK + b * RB, RB)], wsems[j])

    for b in range(nb):
        i = b % 3
        if wbacks[i] is not None:
            wbacks[i].wait()
        gathers[i] = pltpu.async_copy(
            ys_hbm.at[posbuf.at[pl.ds(b * RB, RB)]], bufs[i], gsems[i])
        if b > 0:
            wb(b - 1)
    wb(nb - 1)
    for j in range(3):
        if wbacks[j] is not None:
            wbacks[j].wait()


def _unpermute_sc(ys, pos):
    kfn = pl.kernel(
        _unperm_body, mesh=_sc_mesh(),
        compiler_params=pltpu.CompilerParams(needs_layout_passes=False),
        out_type=jax.ShapeDtypeStruct((N_TOK, OUT_FEAT), jnp.float32),
        scratch_types=[
            pltpu.VMEM((CHUNK,), jnp.int32),
            pltpu.VMEM((RB, OUT_FEAT), jnp.float32),
            pltpu.VMEM((RB, OUT_FEAT), jnp.float32),
            pltpu.VMEM((RB, OUT_FEAT), jnp.float32),
            pltpu.SemaphoreType.DMA,
            pltpu.SemaphoreType.DMA,
            pltpu.SemaphoreType.DMA,
            pltpu.SemaphoreType.DMA,
            pltpu.SemaphoreType.DMA,
            pltpu.SemaphoreType.DMA,
        ],
    )
    return kfn(ys, pos)


def kernel(inp, gate, weight):
    xs, pos, gids, ntiles = _route_sc(inp, gate)
    ys = _grouped_matmul(xs, weight, gids, ntiles)
    return _unpermute_sc(ys, pos)
